# baseline (speedup 1.0000x reference)
import jax
import jax.numpy as jnp
from jax.experimental import pallas as pl


def _params_body(z_ref, gw_ref, bw_ref, bg_ref, bb_ref, bm_ref, bv_ref,
                 scale_ref, bias_ref):
    eps = 1e-3
    a = bg_ref[...] * jax.lax.rsqrt(bv_ref[...] + eps)
    c0 = bb_ref[...] - bm_ref[...] * a
    g = jnp.dot(z_ref[...], gw_ref[...],
                preferred_element_type=jnp.float32)
    bb = jnp.dot(z_ref[...], bw_ref[...],
                 preferred_element_type=jnp.float32)
    onepg = 1.0 + g
    scale = a * onepg
    bias = c0 * onepg + bb
    scale_ref[...] = jnp.broadcast_to(scale[:, None, :], scale_ref.shape)
    bias_ref[...] = jnp.broadcast_to(bias[:, None, :], bias_ref.shape)


def _apply_body(scale_ref, bias_ref, x_ref, o_ref):
    o_ref[...] = x_ref[...] * scale_ref[...] + bias_ref[...]


def kernel(x, z, bn_gamma, bn_beta, bn_mean, bn_var, gamma_w, beta_w):
    B, H, W, C = x.shape
    WC = W * C
    BH = 56

    scale3, bias3 = pl.pallas_call(
        _params_body,
        out_shape=[
            jax.ShapeDtypeStruct((B, W, C), jnp.float32),
            jax.ShapeDtypeStruct((B, W, C), jnp.float32),
        ],
    )(z, gamma_w, beta_w,
      bn_gamma.reshape(1, C), bn_beta.reshape(1, C),
      bn_mean.reshape(1, C), bn_var.reshape(1, C))

    scale2 = scale3.reshape(B, 1, WC)
    bias2 = bias3.reshape(B, 1, WC)
    x2 = x.reshape(B, H, WC)

    out2 = pl.pallas_call(
        _apply_body,
        grid=(B, H // BH),
        in_specs=[
            pl.BlockSpec((1, 1, WC), lambda b, h: (b, 0, 0)),
            pl.BlockSpec((1, 1, WC), lambda b, h: (b, 0, 0)),
            pl.BlockSpec((1, BH, WC), lambda b, h: (b, h, 0)),
        ],
        out_specs=pl.BlockSpec((1, BH, WC), lambda b, h: (b, h, 0)),
        out_shape=jax.ShapeDtypeStruct((B, H, WC), jnp.float32),
    )(scale2, bias2, x2)

    return out2.reshape(B, H, W, C)


# speedup vs baseline: 4.1349x; 4.1349x over previous
import jax
import jax.numpy as jnp
from jax.experimental import pallas as pl


def _params_body(z_ref, gw_ref, bw_ref, bg_ref, bb_ref, bm_ref, bv_ref,
                 scale_ref, bias_ref):
    eps = 1e-3
    a = bg_ref[...] * jax.lax.rsqrt(bv_ref[...] + eps)
    c0 = bb_ref[...] - bm_ref[...] * a
    g = jnp.dot(z_ref[...], gw_ref[...],
                preferred_element_type=jnp.float32)
    bb = jnp.dot(z_ref[...], bw_ref[...],
                 preferred_element_type=jnp.float32)
    onepg = 1.0 + g
    scale = a * onepg
    bias = c0 * onepg + bb
    scale_ref[...] = jnp.broadcast_to(scale[:, None, :], scale_ref.shape)
    bias_ref[...] = jnp.broadcast_to(bias[:, None, :], bias_ref.shape)


def _apply_body(scale_ref, bias_ref, x_ref, o_ref):
    C = x_ref.shape[-1]
    s = scale_ref[...][:, :1, :].reshape(1, 1, 1, C)
    t = bias_ref[...][:, :1, :].reshape(1, 1, 1, C)
    o_ref[...] = x_ref[...] * s + t


def kernel(x, z, bn_gamma, bn_beta, bn_mean, bn_var, gamma_w, beta_w):
    B, H, W, C = x.shape
    BH = 56

    scale3, bias3 = pl.pallas_call(
        _params_body,
        out_shape=[
            jax.ShapeDtypeStruct((B, 8, C), jnp.float32),
            jax.ShapeDtypeStruct((B, 8, C), jnp.float32),
        ],
    )(z, gamma_w, beta_w,
      bn_gamma.reshape(1, C), bn_beta.reshape(1, C),
      bn_mean.reshape(1, C), bn_var.reshape(1, C))

    out = pl.pallas_call(
        _apply_body,
        grid=(B, H // BH),
        in_specs=[
            pl.BlockSpec((1, 8, C), lambda b, h: (b, 0, 0)),
            pl.BlockSpec((1, 8, C), lambda b, h: (b, 0, 0)),
            pl.BlockSpec((1, BH, W, C), lambda b, h: (b, h, 0, 0)),
        ],
        out_specs=pl.BlockSpec((1, BH, W, C), lambda b, h: (b, h, 0, 0)),
        out_shape=jax.ShapeDtypeStruct((B, H, W, C), jnp.float32),
    )(scale3, bias3, x)

    return out


# speedup vs baseline: 16.6594x; 4.0290x over previous
import jax
import jax.numpy as jnp
from jax import lax
from jax.experimental import pallas as pl


def _params_body(z_ref, gwt_ref, bwt_ref, bn_ref, scale_ref, bias_ref):
    eps = 1e-3
    bg = bn_ref[0:1, :]
    bb = bn_ref[1:2, :]
    bm = bn_ref[2:3, :]
    bv = bn_ref[3:4, :]
    a = bg * lax.rsqrt(bv + eps)
    c0 = bb - bm * a
    m = jnp.concatenate([a, c0], axis=0)
    C = m.shape[1]
    eye = (lax.broadcasted_iota(jnp.int32, (C, C), 0)
           == lax.broadcasted_iota(jnp.int32, (C, C), 1)).astype(jnp.float32)
    mt = lax.dot_general(eye, m, (((1,), (1,)), ((), ())),
                         preferred_element_type=jnp.float32)
    a_col = mt[:, 0:1]
    c0_col = mt[:, 1:2]
    g_t = lax.dot_general(gwt_ref[...], z_ref[...],
                          (((1,), (1,)), ((), ())),
                          preferred_element_type=jnp.float32)
    bb_t = lax.dot_general(bwt_ref[...], z_ref[...],
                           (((1,), (1,)), ((), ())),
                           preferred_element_type=jnp.float32)
    onepg = 1.0 + g_t
    scale_t = a_col * onepg
    bias_t = c0_col * onepg + bb_t
    B = scale_t.shape[1]
    for b in range(B):
        scale_ref[b] = scale_t[:, b:b + 1]
        bias_ref[b] = bias_t[:, b:b + 1]


def _apply_body(scale_ref, bias_ref, xt_ref, o_ref):
    C = xt_ref.shape[2]
    s = scale_ref[...].reshape(1, 1, C, 1)
    t = bias_ref[...].reshape(1, 1, C, 1)
    o_ref[...] = xt_ref[...] * s + t


def kernel(x, z, bn_gamma, bn_beta, bn_mean, bn_var, gamma_w, beta_w):
    B, H, W, C = x.shape
    BH = 56

    xt = jnp.transpose(x, (0, 1, 3, 2))
    gw_t = gamma_w.T
    bw_t = beta_w.T
    bn = jnp.stack([bn_gamma, bn_beta, bn_mean, bn_var])

    scale_c, bias_c = pl.pallas_call(
        _params_body,
        out_shape=[
            jax.ShapeDtypeStruct((B, C, 1), jnp.float32),
            jax.ShapeDtypeStruct((B, C, 1), jnp.float32),
        ],
    )(z, gw_t, bw_t, bn)

    out_t = pl.pallas_call(
        _apply_body,
        grid=(B, H // BH),
        in_specs=[
            pl.BlockSpec((1, C, 1), lambda b, h: (b, 0, 0)),
            pl.BlockSpec((1, C, 1), lambda b, h: (b, 0, 0)),
            pl.BlockSpec((1, BH, C, W), lambda b, h: (b, h, 0, 0)),
        ],
        out_specs=pl.BlockSpec((1, BH, C, W), lambda b, h: (b, h, 0, 0)),
        out_shape=jax.ShapeDtypeStruct((B, H, C, W), jnp.float32),
    )(scale_c, bias_c, xt)

    return jnp.transpose(out_t, (0, 1, 3, 2))
